# baseline (speedup 1.0000x reference)
import jax
import jax.numpy as jnp
from jax import lax
from jax.experimental import pallas as pl
from jax.experimental.pallas import tpu as pltpu
from jax.experimental.pallas import tpu_sc as plsc

N = 3276800
D = 2
N_CATEGORIES = 1000000
NUM_CORES = 2
NUM_SUBCORES = 16
NUM_WORKERS = NUM_CORES * NUM_SUBCORES
LANES = 16

PER_WORKER = N // NUM_WORKERS
CHUNK = 10240
NCHUNK = PER_WORKER // CHUNK


def _sc_body(cat_hbm, mu_hbm, mu_out_hbm, var_hbm,
             idx0, idx1, vals0, vals1, var0, var1, sg0, sg1, sidx):
    wid = lax.axis_index("s") * NUM_CORES + lax.axis_index("c")
    worker_base = wid * PER_WORKER

    idx = (idx0, idx1)
    vals = (vals0, vals1)
    varb = (var0, var1)
    sg = (sg0, sg1)

    def load_idx(ci, b):
        pltpu.sync_copy(cat_hbm.at[pl.ds(worker_base + ci * CHUNK, CHUNK)],
                        idx[b])

    def start_gather(b):
        pltpu.async_copy(mu_hbm.at[idx[b]], vals[b], sg[b])

    def finish_chunk(ci, b):
        pltpu.make_async_copy(mu_hbm.at[idx[b]], vals[b], sg[b]).wait()

        def vec_body(j, carry):
            v = vals[b][pl.ds(j * LANES, LANES)]
            varb[b][pl.ds(j * LANES, LANES)] = v - v * v
            return carry

        lax.fori_loop(0, CHUNK // LANES, vec_body, 0, unroll=4)
        base = worker_base + ci * CHUNK
        pltpu.sync_copy(vals[b], mu_out_hbm.at[pl.ds(base, CHUNK)])
        pltpu.sync_copy(varb[b], var_hbm.at[pl.ds(base, CHUNK)])

    load_idx(0, 0)
    start_gather(0)

    def pair_body(k, carry):
        ci0 = 2 * k
        load_idx(ci0 + 1, 1)
        start_gather(1)
        finish_chunk(ci0, 0)

        @pl.when(k < NCHUNK // 2 - 1)
        def _():
            load_idx(ci0 + 2, 0)
            start_gather(0)

        finish_chunk(ci0 + 1, 1)
        return carry

    lax.fori_loop(0, NCHUNK // 2, pair_body, 0)


@jax.jit
def kernel(Xnew, mu):
    cat = Xnew[:, -1].astype(jnp.int32)
    mesh = plsc.VectorSubcoreMesh(
        core_axis_name="c", subcore_axis_name="s",
        num_cores=NUM_CORES, num_subcores=NUM_SUBCORES,
    )
    run = pl.kernel(
        _sc_body,
        out_type=(
            jax.ShapeDtypeStruct((N,), jnp.float32),
            jax.ShapeDtypeStruct((N,), jnp.float32),
        ),
        mesh=mesh,
        scratch_types=[
            pltpu.VMEM((CHUNK,), jnp.int32),
            pltpu.VMEM((CHUNK,), jnp.int32),
            pltpu.VMEM((CHUNK,), jnp.float32),
            pltpu.VMEM((CHUNK,), jnp.float32),
            pltpu.VMEM((CHUNK,), jnp.float32),
            pltpu.VMEM((CHUNK,), jnp.float32),
            pltpu.SemaphoreType.DMA,
            pltpu.SemaphoreType.DMA,
            pltpu.SemaphoreType.DMA,
        ],
    )
    mu_out, var = run(cat, mu)
    return (mu_out, var)


# speedup vs baseline: 2.1052x; 2.1052x over previous
import jax
import jax.numpy as jnp
from jax import lax
from jax.experimental import pallas as pl
from jax.experimental.pallas import tpu as pltpu
from jax.experimental.pallas import tpu_sc as plsc

N = 3276800
D = 2
N_CATEGORIES = 1000000
NUM_CORES = 2
NUM_SUBCORES = 16
NUM_WORKERS = NUM_CORES * NUM_SUBCORES
LANES = 16

PER_WORKER = N // NUM_WORKERS
CHUNK = 10240
NCHUNK = PER_WORKER // CHUNK


STAGE = N_CATEGORIES // 8
SCHUNK = 25000


def _sc_body(cat_hbm, mu_hbm, mu_out_hbm, var_hbm,
             idx0, idx1, vals0, vals1, stage_v, mu_sp,
             sg0, sg1, sidx):
    sid = lax.axis_index("s")
    wid = sid * NUM_CORES + lax.axis_index("c")
    worker_base = wid * PER_WORKER

    @pl.when(sid < 8)
    def _():
        def stage_body(k, carry):
            off = sid * STAGE + k * SCHUNK
            pltpu.sync_copy(mu_hbm.at[pl.ds(off, SCHUNK)], stage_v)
            pltpu.sync_copy(stage_v, mu_sp.at[pl.ds(off, SCHUNK)])
            return carry

        lax.fori_loop(0, STAGE // SCHUNK, stage_body, 0)

    plsc.subcore_barrier()

    idx = (idx0, idx1)
    vals = (vals0, vals1)
    sg = (sg0, sg1)

    def load_idx(ci, b):
        pltpu.sync_copy(cat_hbm.at[pl.ds(worker_base + ci * CHUNK, CHUNK)],
                        idx[b])

    def start_gather(b):
        pltpu.async_copy(mu_sp.at[idx[b]], vals[b], sg[b])

    def finish_chunk(ci, b):
        pltpu.make_async_copy(mu_sp.at[idx[b]], vals[b], sg[b]).wait()
        base = worker_base + ci * CHUNK
        pltpu.sync_copy(vals[b], mu_out_hbm.at[pl.ds(base, CHUNK)])

        def vec_body(j, carry):
            v = vals[b][pl.ds(j * LANES, LANES)]
            vals[b][pl.ds(j * LANES, LANES)] = v - v * v
            return carry

        lax.fori_loop(0, CHUNK // LANES, vec_body, 0, unroll=4)
        pltpu.sync_copy(vals[b], var_hbm.at[pl.ds(base, CHUNK)])

    load_idx(0, 0)
    start_gather(0)

    def pair_body(k, carry):
        ci0 = 2 * k
        load_idx(ci0 + 1, 1)
        start_gather(1)
        finish_chunk(ci0, 0)

        @pl.when(k < NCHUNK // 2 - 1)
        def _():
            load_idx(ci0 + 2, 0)
            start_gather(0)

        finish_chunk(ci0 + 1, 1)
        return carry

    lax.fori_loop(0, NCHUNK // 2, pair_body, 0)


@jax.jit
def kernel(Xnew, mu):
    cat = Xnew[:, -1].astype(jnp.int32)
    mesh = plsc.VectorSubcoreMesh(
        core_axis_name="c", subcore_axis_name="s",
        num_cores=NUM_CORES, num_subcores=NUM_SUBCORES,
    )
    run = pl.kernel(
        _sc_body,
        out_type=(
            jax.ShapeDtypeStruct((N,), jnp.float32),
            jax.ShapeDtypeStruct((N,), jnp.float32),
        ),
        mesh=mesh,
        scratch_types=[
            pltpu.VMEM((CHUNK,), jnp.int32),
            pltpu.VMEM((CHUNK,), jnp.int32),
            pltpu.VMEM((CHUNK,), jnp.float32),
            pltpu.VMEM((CHUNK,), jnp.float32),
            pltpu.VMEM((SCHUNK,), jnp.float32),
            pltpu.VMEM_SHARED((N_CATEGORIES,), jnp.float32),
            pltpu.SemaphoreType.DMA,
            pltpu.SemaphoreType.DMA,
            pltpu.SemaphoreType.DMA,
        ],
    )
    mu_out, var = run(cat, mu)
    return (mu_out, var)
